# baseline (device time: 28415 ns/iter reference)
import jax
import jax.numpy as jnp
from jax import lax
from jax.experimental import pallas as pl
from jax.experimental.pallas import tpu as pltpu

N_DEV = 16
M = 1024
K = 512
N = 512

STREAMS = ((0, 640, (1, 4)), (640, 384, (4, 1)))
N_EXCH = 24
RS_ROWS = 3 * (160 + 40 + 96 + 24)


def kernel(t, W):
    def body(t_ref, w_ref, out_ref, stage_ref, comm_ref, ag_ref,
             send_sems, recv_sems):
        i = lax.axis_index("i")

        def group(u):
            g = lax.rem(lax.div(i, u), 4)
            return g, i - g * u

        barrier_sem = pltpu.get_barrier_semaphore()
        for u in (1, 4):
            g, gbase = group(u)
            for d in (1, 2, 3):
                peer = gbase + lax.rem(g + d, 4) * u
                pl.semaphore_signal(
                    barrier_sem, inc=1,
                    device_id=(peer,),
                    device_id_type=pl.DeviceIdType.MESH,
                )
        pl.semaphore_wait(barrier_sem, 6)

        def mm(row_lo, rows):
            out_ref[pl.ds(row_lo, rows), :] = jnp.dot(
                t_ref[pl.ds(row_lo, rows), :], w_ref[...],
                preferred_element_type=jnp.float32,
            )

        lo = [jnp.int32(base) for base, _, _ in STREAMS]
        pending = [None, None]
        ctr = {"sem": 0, "comm": 0, "stage": 0}

        def start(s, slot):
            _, R, units = STREAMS[s]
            sem_base = ctr["sem"]
            ctr["sem"] += 3
            if slot < 2:
                H = R // 4 if slot == 0 else R // 16
                u = units[slot]
                g, gbase = group(u)
                keep_lo = lo[s] + g * H
                comm_base = ctr["comm"]
                ctr["comm"] += 3 * H
                rdmas = []
                for d in (1, 2, 3):
                    jm = lax.rem(g + d, 4)
                    peer = gbase + jm * u
                    src_off = ctr["stage"]
                    ctr["stage"] += H
                    send_lo = lo[s] + jm * H
                    stage_ref[pl.ds(src_off, H), :] = out_ref[
                        pl.ds(send_lo, H), :
                    ].astype(jnp.bfloat16)
                    r = 4 - d
                    rdma = pltpu.make_async_remote_copy(
                        src_ref=stage_ref.at[pl.ds(src_off, H), :],
                        dst_ref=comm_ref.at[
                            pl.ds(comm_base + (r - 1) * H, H), :
                        ],
                        send_sem=send_sems.at[sem_base + r - 1],
                        recv_sem=recv_sems.at[sem_base + r - 1],
                        device_id=(peer,),
                        device_id_type=pl.DeviceIdType.MESH,
                    )
                    rdma.start()
                    rdmas.append(rdma)
                lo[s] = keep_lo
                pending[s] = (rdmas, keep_lo, comm_base, slot, H)
            else:
                H = R // 16 if slot == 2 else R // 4
                u = units[3 - slot]
                g, gbase = group(u)
                rdmas = []
                for d in (1, 2, 3):
                    peer = gbase + lax.rem(g + d, 4) * u
                    r = 4 - d
                    rdma = pltpu.make_async_remote_copy(
                        src_ref=ag_ref.at[pl.ds(lo[s], H), :],
                        dst_ref=ag_ref.at[pl.ds(lo[s], H), :],
                        send_sem=send_sems.at[sem_base + r - 1],
                        recv_sem=recv_sems.at[sem_base + r - 1],
                        device_id=(peer,),
                        device_id_type=pl.DeviceIdType.MESH,
                    )
                    rdma.start()
                    rdmas.append(rdma)
                pending[s] = (rdmas, lo[s] - g * H, g, slot, H)
                lo[s] = lo[s] - g * H

        def finish(s):
            rdmas, aux, aux2, slot, H = pending[s]
            for rdma in rdmas:
                rdma.wait()
            if slot < 2:
                keep_lo, comm_base = aux, aux2
                out_ref[pl.ds(keep_lo, H), :] += (
                    comm_ref[pl.ds(comm_base, H), :].astype(jnp.float32)
                    + comm_ref[pl.ds(comm_base + H, H), :].astype(
                        jnp.float32
                    )
                    + comm_ref[pl.ds(comm_base + 2 * H, H), :].astype(
                        jnp.float32
                    )
                )
                if slot == 1:
                    ag_ref[pl.ds(keep_lo, H), :] = out_ref[
                        pl.ds(keep_lo, H), :
                    ].astype(jnp.bfloat16)
            else:
                block_lo, g = aux, aux2
                if slot == 2:
                    out_ref[pl.ds(block_lo, 4 * H), :] = ag_ref[
                        pl.ds(block_lo, 4 * H), :
                    ].astype(jnp.float32)
                else:
                    for d in (1, 2, 3):
                        jm = lax.rem(g + d, 4)
                        out_ref[pl.ds(block_lo + jm * H, H), :] = ag_ref[
                            pl.ds(block_lo + jm * H, H), :
                        ].astype(jnp.float32)

        def start0(s, d, sem_base, comm_base, stage_base, ginfo):
            base, R, units = STREAMS[s]
            H = R // 4
            g, gbase = ginfo
            jm = lax.rem(g + d, 4)
            peer = gbase + jm * u0(s)
            send_lo = base + jm * H
            mm(send_lo, H)
            stage_ref[pl.ds(stage_base + (d - 1) * H, H), :] = out_ref[
                pl.ds(send_lo, H), :
            ].astype(jnp.bfloat16)
            r = 4 - d
            rdma = pltpu.make_async_remote_copy(
                src_ref=stage_ref.at[pl.ds(stage_base + (d - 1) * H, H), :],
                dst_ref=comm_ref.at[pl.ds(comm_base + (r - 1) * H, H), :],
                send_sem=send_sems.at[sem_base + r - 1],
                recv_sem=recv_sems.at[sem_base + r - 1],
                device_id=(peer,),
                device_id_type=pl.DeviceIdType.MESH,
            )
            rdma.start()
            return rdma

        def u0(s):
            return STREAMS[s][2][0]

        slot0 = []
        for s in range(2):
            base, R, units = STREAMS[s]
            H = R // 4
            g, gbase = group(u0(s))
            sem_base = ctr["sem"]
            ctr["sem"] += 3
            comm_base = ctr["comm"]
            ctr["comm"] += 3 * H
            stage_base = ctr["stage"]
            ctr["stage"] += 3 * H
            slot0.append((sem_base, comm_base, stage_base, (g, gbase), []))
        for d in (1, 2, 3):
            for s in range(2):
                sem_base, comm_base, stage_base, ginfo, rdmas = slot0[s]
                rdmas.append(
                    start0(s, d, sem_base, comm_base, stage_base, ginfo)
                )
        for s in range(2):
            base, R, units = STREAMS[s]
            H = R // 4
            sem_base, comm_base, stage_base, (g, gbase), rdmas = slot0[s]
            keep_lo = base + g * H
            mm(keep_lo, H)
            lo[s] = keep_lo
            pending[s] = (rdmas, keep_lo, comm_base, 0, H)

        for slot in range(1, 4):
            for s in range(2):
                finish(s)
                start(s, slot)
        for s in range(2):
            finish(s)

    return pl.pallas_call(
        body,
        out_shape=jax.ShapeDtypeStruct((M, N), jnp.float32),
        in_specs=[
            pl.BlockSpec(memory_space=pltpu.VMEM),
            pl.BlockSpec(memory_space=pltpu.VMEM),
        ],
        out_specs=pl.BlockSpec(memory_space=pltpu.VMEM),
        scratch_shapes=[
            pltpu.VMEM((RS_ROWS, N), jnp.bfloat16),
            pltpu.VMEM((RS_ROWS, N), jnp.bfloat16),
            pltpu.VMEM((M, N), jnp.bfloat16),
            pltpu.SemaphoreType.DMA((N_EXCH,)),
            pltpu.SemaphoreType.DMA((N_EXCH,)),
        ],
        compiler_params=pltpu.CompilerParams(collective_id=0),
    )(t, W)
